# baseline (speedup 1.0000x reference)
import functools

import jax
import jax.numpy as jnp
from jax import lax
from jax.experimental import pallas as pl
from jax.experimental.pallas import tpu as pltpu
from jax.experimental.pallas import tpu_sc as plsc

N = 10000
E = 160000
FIN = 128
D = 32
B = 64

NC = 2
NS = 16
NW = NC * NS
CH = 128
CPW = 40
EPW = CH * CPW
EPAD = EPW * NW
NTBL = 10112
RPT = NTBL // NS
WP = 128

_F32 = jnp.float32


GK = 4


def _gather_body(table_hbm, idx_hbm, out_hbm, idx_v, rows_v, gsem, wsem):
    cid = lax.axis_index("c")
    sid = lax.axis_index("s")
    wid = cid * NS + sid
    pltpu.sync_copy(idx_hbm.at[wid], idx_v)

    def quad(q, c):
        j = q * GK
        for b in range(GK):
            pltpu.async_copy(table_hbm.at[idx_v.at[j + b]], rows_v.at[b], gsem)
        for b in range(GK):
            pltpu.make_async_copy(
                table_hbm.at[idx_v.at[j + b]], rows_v.at[b], gsem).wait()
        for b in range(GK):
            pltpu.async_copy(
                rows_v.at[b], out_hbm.at[pl.ds(wid * EPW + (j + b) * CH, CH)],
                wsem)
        for b in range(GK):
            pltpu.make_async_copy(
                rows_v.at[b], out_hbm.at[pl.ds(wid * EPW + (j + b) * CH, CH)],
                wsem).wait()
        return c

    lax.fori_loop(0, CPW // GK, quad, 0)


@functools.cache
def _gather():
    return pl.kernel(
        _gather_body,
        out_type=jax.ShapeDtypeStruct((EPAD, WP), _F32),
        mesh=plsc.VectorSubcoreMesh(core_axis_name="c", subcore_axis_name="s"),
        scratch_types=[
            pltpu.VMEM((CPW, CH), jnp.int32),
            pltpu.VMEM((GK, CH, WP), _F32),
            pltpu.SemaphoreType.DMA,
            pltpu.SemaphoreType.DMA,
        ],
    )


def _make_scatter():
    SK = 2

    def body(idx_hbm, src_hbm, zero_hbm, out_hbm, idx_v, rows_v, table, sem):
        cid = lax.axis_index("c")
        sid = lax.axis_index("s")
        wid = cid * NS + sid
        pltpu.sync_copy(idx_hbm.at[wid], idx_v)
        pltpu.sync_copy(zero_hbm, table.at[pl.ds(sid * RPT, RPT)])
        plsc.subcore_barrier()

        def pair(q, c):
            j = q * SK
            for b in range(SK):
                pltpu.async_copy(
                    src_hbm.at[pl.ds(wid * EPW + (j + b) * CH, CH)],
                    rows_v.at[b], sem)
            for b in range(SK):
                pltpu.make_async_copy(
                    src_hbm.at[pl.ds(wid * EPW + (j + b) * CH, CH)],
                    rows_v.at[b], sem).wait()
            for b in range(SK):
                pltpu.sync_copy(rows_v.at[b], table.at[idx_v.at[j + b]],
                                add=True)
            return c

        lax.fori_loop(0, CPW // SK, pair, 0)
        plsc.subcore_barrier()
        pltpu.sync_copy(table.at[pl.ds(sid * RPT, RPT)],
                        out_hbm.at[cid, pl.ds(sid * RPT, RPT)])

    return pl.kernel(
        body,
        out_type=jax.ShapeDtypeStruct((NC, NTBL, WP), _F32),
        mesh=plsc.VectorSubcoreMesh(core_axis_name="c", subcore_axis_name="s"),
        scratch_types=[
            pltpu.VMEM((CPW, CH), jnp.int32),
            pltpu.VMEM((2, CH, WP), _F32),
            pltpu.VMEM_SHARED((NTBL, WP), _F32),
            pltpu.SemaphoreType.DMA,
        ],
    )


_make_scatter = functools.cache(_make_scatter)


def _lin0_body(x_ref, w_ref, b_ref, o_ref):
    v = jnp.maximum(
        jnp.dot(x_ref[...], w_ref[...], preferred_element_type=_F32)
        + b_ref[...][None, :], 0.0)
    o_ref[...] = jnp.concatenate([v, jnp.zeros((N, WP - D), _F32)], axis=1)


_lin0 = pl.pallas_call(
    _lin0_body, out_shape=jax.ShapeDtypeStruct((N, WP), _F32))


BLK = 512


def _msg_body(ea_ref, xj_ref, w1_ref, b1_ref, w2_ref, b2_ref, o_ref):
    h2 = jnp.maximum(
        jnp.dot(ea_ref[...], w1_ref[...], preferred_element_type=_F32)
        + b1_ref[...][None, :], 0.0)
    w = (jnp.dot(h2, w2_ref[...], preferred_element_type=_F32)
         + b2_ref[...][None, :])
    rep = (lax.broadcasted_iota(jnp.int32, (D, D * D), 1) // D
           == lax.broadcasted_iota(jnp.int32, (D, D * D), 0)).astype(_F32)
    red = (lax.broadcasted_iota(jnp.int32, (D * D, D), 0) % D
           == lax.broadcasted_iota(jnp.int32, (D * D, D), 1)).astype(_F32)
    xr = jnp.dot(xj_ref[...][:, :D], rep, preferred_element_type=_F32)
    msg = jnp.dot(xr * w, red, preferred_element_type=_F32)
    o_ref[...] = jnp.concatenate(
        [msg, jnp.ones((BLK, 1), _F32), jnp.zeros((BLK, WP - D - 1), _F32)],
        axis=1)


_msg = pl.pallas_call(
    _msg_body,
    grid=(EPAD // BLK,),
    in_specs=[
        pl.BlockSpec((BLK, 5), lambda i: (i, 0)),
        pl.BlockSpec((BLK, WP), lambda i: (i, 0)),
        pl.BlockSpec((5, FIN), lambda i: (0, 0)),
        pl.BlockSpec((FIN,), lambda i: (0,)),
        pl.BlockSpec((FIN, D * D), lambda i: (0, 0)),
        pl.BlockSpec((D * D,), lambda i: (0,)),
    ],
    out_specs=pl.BlockSpec((BLK, WP), lambda i: (i, 0)),
    out_shape=jax.ShapeDtypeStruct((EPAD, WP), _F32),
)


def _upd_body(p_ref, dg_ref, h_ref, cb_ref, wih_ref, whh_ref, bih_ref,
              bhh_ref, o_ref):
    p = p_ref[...]
    dg = dg_ref[...]
    agg = p[0, :N, :D] + p[1, :N, :D]
    deg = jnp.maximum(dg[0, :N, D:D + 1] + dg[1, :N, D:D + 1], 1.0)
    m = jnp.maximum(agg / deg + cb_ref[...][None, :], 0.0)
    h = h_ref[...][:, :D]
    gi = lax.dot_general(m, wih_ref[...], (((1,), (1,)), ((), ())),
                         preferred_element_type=_F32) + bih_ref[...][None, :]
    gh = lax.dot_general(h, whh_ref[...], (((1,), (1,)), ((), ())),
                         preferred_element_type=_F32) + bhh_ref[...][None, :]
    r = jax.nn.sigmoid(gi[:, 0:D] + gh[:, 0:D])
    z = jax.nn.sigmoid(gi[:, D:2 * D] + gh[:, D:2 * D])
    n = jnp.tanh(gi[:, 2 * D:3 * D] + r * gh[:, 2 * D:3 * D])
    v = (1.0 - z) * n + z * h
    o_ref[...] = jnp.concatenate([v, jnp.zeros((N, WP - D), _F32)], axis=1)


_upd = pl.pallas_call(
    _upd_body, out_shape=jax.ShapeDtypeStruct((N, WP), _F32))


def _s2s_body(out_ref, bat_ref, wih_ref, whh_ref, bih_ref, bhh_ref, o_ref):
    outv = out_ref[...][:, :D]
    onehot = (bat_ref[...] == lax.broadcasted_iota(jnp.int32, (N, B), 1)
              ).astype(_F32)
    wih = wih_ref[...]
    whh = whh_ref[...]
    bih = bih_ref[...]
    bhh = bhh_ref[...]
    qs = jnp.zeros((B, 2 * D), _F32)
    hh = jnp.zeros((B, D), _F32)
    cc = jnp.zeros((B, D), _F32)
    for _ in range(3):
        gates = (lax.dot_general(qs, wih, (((1,), (1,)), ((), ())),
                                 preferred_element_type=_F32) + bih[None, :]
                 + lax.dot_general(hh, whh, (((1,), (1,)), ((), ())),
                                   preferred_element_type=_F32) + bhh[None, :])
        ig = jax.nn.sigmoid(gates[:, 0:D])
        fg = jax.nn.sigmoid(gates[:, D:2 * D])
        gg = jnp.tanh(gates[:, 2 * D:3 * D])
        og = jax.nn.sigmoid(gates[:, 3 * D:4 * D])
        cc = fg * cc + ig * gg
        hh = og * jnp.tanh(cc)
        qb = jnp.dot(onehot, hh, preferred_element_type=_F32)
        e = jnp.sum(outv * qb, axis=1, keepdims=True)
        masked = jnp.where(onehot > 0.0, e, -jnp.inf)
        emax = jnp.max(masked, axis=0, keepdims=True)
        emax = jnp.where(jnp.isfinite(emax), emax, 0.0)
        emax_n = lax.dot_general(onehot, emax, (((1,), (1,)), ((), ())),
                                 preferred_element_type=_F32)
        a = jnp.exp(e - emax_n)
        asum = lax.dot_general(onehot, a, (((0,), (0,)), ((), ())),
                               preferred_element_type=_F32)
        asum_n = jnp.dot(onehot, asum, preferred_element_type=_F32)
        a = a / (asum_n + 1e-16)
        r_ = lax.dot_general(onehot, a * outv, (((0,), (0,)), ((), ())),
                             preferred_element_type=_F32)
        qs = jnp.concatenate([hh, r_], axis=1)
    o_ref[...] = qs


_s2s = pl.pallas_call(
    _s2s_body, out_shape=jax.ShapeDtypeStruct((B, 2 * D), _F32))


def kernel(x, edge_index, edge_attr, batch, lin0_w, lin0_b, mlp_w1, mlp_b1,
           mlp_w2, mlp_b2, conv_bias, gru_w_ih, gru_w_hh, gru_b_ih, gru_b_hh,
           lstm_w_ih, lstm_w_hh, lstm_b_ih, lstm_b_hh):
    src = edge_index[0].astype(jnp.int32)
    dst = edge_index[1].astype(jnp.int32)
    pad = EPAD - E
    src_r = jnp.concatenate([src, jnp.zeros((pad,), jnp.int32)]
                            ).reshape(NW, CPW, CH)
    sent = N + jnp.arange(pad, dtype=jnp.int32) % (NTBL - N)
    dst_r = jnp.concatenate([dst, sent]).reshape(NW, CPW, CH)
    ea_p = jnp.concatenate(
        [edge_attr.astype(_F32), jnp.zeros((pad, edge_attr.shape[1]), _F32)])
    zeros_hbm = jnp.zeros((RPT, WP), _F32)

    out = _lin0(x, lin0_w, lin0_b)
    deg_p = None
    for _ in range(3):
        xj = _gather()(out, src_r)
        msg = _msg(ea_p, xj, mlp_w1, mlp_b1, mlp_w2, mlp_b2)
        part = _make_scatter()(dst_r, msg, zeros_hbm)
        if deg_p is None:
            deg_p = part
        out = _upd(part, deg_p, out, conv_bias, gru_w_ih, gru_w_hh,
                   gru_b_ih, gru_b_hh)
    bat2 = batch.astype(jnp.int32).reshape(N, 1)
    return _s2s(out, bat2, lstm_w_ih, lstm_w_hh, lstm_b_ih, lstm_b_hh)


# speedup vs baseline: 1.3127x; 1.3127x over previous
import functools

import jax
import jax.numpy as jnp
from jax import lax
from jax.experimental import pallas as pl
from jax.experimental.pallas import tpu as pltpu
from jax.experimental.pallas import tpu_sc as plsc

N = 10000
E = 160000
FIN = 128
D = 32
B = 64

NC = 2
NS = 16
NW = NC * NS
CH = 128
CPW = 40
EPW = CH * CPW
EPAD = EPW * NW
NTBL = 10112
RPT = NTBL // NS
WP = 128

_F32 = jnp.float32


GK = 4


@functools.cache
def _gather(cpw):
    epw = cpw * CH

    def body(table_hbm, idx_hbm, out_hbm, idx_v, rows_v, gsem, wsem):
        cid = lax.axis_index("c")
        sid = lax.axis_index("s")
        wid = cid * NS + sid
        pltpu.sync_copy(idx_hbm.at[wid], idx_v)

        def quad(q, c):
            j = q * GK
            for b in range(GK):
                pltpu.async_copy(table_hbm.at[idx_v.at[j + b]], rows_v.at[b],
                                 gsem)
            for b in range(GK):
                pltpu.make_async_copy(
                    table_hbm.at[idx_v.at[j + b]], rows_v.at[b], gsem).wait()
            for b in range(GK):
                pltpu.async_copy(
                    rows_v.at[b],
                    out_hbm.at[pl.ds(wid * epw + (j + b) * CH, CH)], wsem)
            for b in range(GK):
                pltpu.make_async_copy(
                    rows_v.at[b],
                    out_hbm.at[pl.ds(wid * epw + (j + b) * CH, CH)],
                    wsem).wait()
            return c

        lax.fori_loop(0, cpw // GK, quad, 0)

    return pl.kernel(
        body,
        out_type=jax.ShapeDtypeStruct((epw * NW, WP), _F32),
        mesh=plsc.VectorSubcoreMesh(core_axis_name="c", subcore_axis_name="s"),
        scratch_types=[
            pltpu.VMEM((cpw, CH), jnp.int32),
            pltpu.VMEM((GK, CH, WP), _F32),
            pltpu.SemaphoreType.DMA,
            pltpu.SemaphoreType.DMA,
        ],
    )


@functools.cache
def _make_scatter(cpw, const_rows):
    SK = 2
    epw = cpw * CH

    def body(idx_hbm, src_hbm, zero_hbm, out_hbm, idx_v, rows_v, table, sem):
        cid = lax.axis_index("c")
        sid = lax.axis_index("s")
        wid = cid * NS + sid
        pltpu.sync_copy(idx_hbm.at[wid], idx_v)
        pltpu.sync_copy(zero_hbm, table.at[pl.ds(sid * RPT, RPT)])
        if const_rows:
            pltpu.sync_copy(src_hbm, rows_v.at[0])
        plsc.subcore_barrier()

        if const_rows:
            def chunk(j, c):
                pltpu.sync_copy(rows_v.at[0], table.at[idx_v.at[j]], add=True)
                return c

            lax.fori_loop(0, cpw, chunk, 0)
        else:
            def pair(q, c):
                j = q * SK
                for b in range(SK):
                    pltpu.async_copy(
                        src_hbm.at[pl.ds(wid * epw + (j + b) * CH, CH)],
                        rows_v.at[b], sem)
                for b in range(SK):
                    pltpu.make_async_copy(
                        src_hbm.at[pl.ds(wid * epw + (j + b) * CH, CH)],
                        rows_v.at[b], sem).wait()
                for b in range(SK):
                    pltpu.sync_copy(rows_v.at[b], table.at[idx_v.at[j + b]],
                                    add=True)
                return c

            lax.fori_loop(0, cpw // SK, pair, 0)
        plsc.subcore_barrier()
        pltpu.sync_copy(table.at[pl.ds(sid * RPT, RPT)],
                        out_hbm.at[cid, pl.ds(sid * RPT, RPT)])

    return pl.kernel(
        body,
        out_type=jax.ShapeDtypeStruct((NC, NTBL, WP), _F32),
        mesh=plsc.VectorSubcoreMesh(core_axis_name="c", subcore_axis_name="s"),
        scratch_types=[
            pltpu.VMEM((cpw, CH), jnp.int32),
            pltpu.VMEM((2, CH, WP), _F32),
            pltpu.VMEM_SHARED((NTBL, WP), _F32),
            pltpu.SemaphoreType.DMA,
        ],
    )


def _lin0_body(x_ref, w_ref, b_ref, o_ref):
    v = jnp.maximum(
        jnp.dot(x_ref[...], w_ref[...], preferred_element_type=_F32)
        + b_ref[...][None, :], 0.0)
    o_ref[...] = jnp.concatenate([v, jnp.zeros((N, WP - D), _F32)], axis=1)


_lin0 = pl.pallas_call(
    _lin0_body, out_shape=jax.ShapeDtypeStruct((N, WP), _F32))


BLK = 512


def _msg_body(ea_ref, xj_ref, w1_ref, b1_ref, w2_ref, b2_ref, o_ref):
    h2 = jnp.maximum(
        jnp.dot(ea_ref[...], w1_ref[...], preferred_element_type=_F32)
        + b1_ref[...][None, :], 0.0)
    w = (jnp.dot(h2, w2_ref[...], preferred_element_type=_F32)
         + b2_ref[...][None, :])
    rep = (lax.broadcasted_iota(jnp.int32, (D, D * D), 1) // D
           == lax.broadcasted_iota(jnp.int32, (D, D * D), 0)).astype(_F32)
    red = (lax.broadcasted_iota(jnp.int32, (D * D, D), 0) % D
           == lax.broadcasted_iota(jnp.int32, (D * D, D), 1)).astype(_F32)
    xr = jnp.dot(xj_ref[...][:, :D], rep, preferred_element_type=_F32)
    msg = jnp.dot(xr * w, red, preferred_element_type=_F32)
    o_ref[...] = jnp.concatenate(
        [msg, jnp.zeros((BLK, WP - D), _F32)], axis=1)


@functools.cache
def _msg(nedges):
    return pl.pallas_call(
        _msg_body,
        grid=(nedges // BLK,),
        in_specs=[
            pl.BlockSpec((BLK, 5), lambda i: (i, 0)),
            pl.BlockSpec((BLK, WP), lambda i: (i, 0)),
            pl.BlockSpec((5, FIN), lambda i: (0, 0)),
            pl.BlockSpec((FIN,), lambda i: (0,)),
            pl.BlockSpec((FIN, D * D), lambda i: (0, 0)),
            pl.BlockSpec((D * D,), lambda i: (0,)),
        ],
        out_specs=pl.BlockSpec((BLK, WP), lambda i: (i, 0)),
        out_shape=jax.ShapeDtypeStruct((nedges, WP), _F32),
    )


def _upd_body(p0_ref, p1_ref, dg_ref, h_ref, cb_ref, wih_ref, whh_ref,
              bih_ref, bhh_ref, o_ref):
    p0 = p0_ref[...]
    p1 = p1_ref[...]
    dg = dg_ref[...]
    agg = (p0[0, :N, :D] + p0[1, :N, :D]
           + p1[0, :N, :D] + p1[1, :N, :D])
    deg = jnp.maximum(dg[0, :N, 0:1] + dg[1, :N, 0:1], 1.0)
    m = jnp.maximum(agg / deg + cb_ref[...][None, :], 0.0)
    h = h_ref[...][:, :D]
    gi = lax.dot_general(m, wih_ref[...], (((1,), (1,)), ((), ())),
                         preferred_element_type=_F32) + bih_ref[...][None, :]
    gh = lax.dot_general(h, whh_ref[...], (((1,), (1,)), ((), ())),
                         preferred_element_type=_F32) + bhh_ref[...][None, :]
    r = jax.nn.sigmoid(gi[:, 0:D] + gh[:, 0:D])
    z = jax.nn.sigmoid(gi[:, D:2 * D] + gh[:, D:2 * D])
    n = jnp.tanh(gi[:, 2 * D:3 * D] + r * gh[:, 2 * D:3 * D])
    v = (1.0 - z) * n + z * h
    o_ref[...] = jnp.concatenate([v, jnp.zeros((N, WP - D), _F32)], axis=1)


_upd = pl.pallas_call(
    _upd_body, out_shape=jax.ShapeDtypeStruct((N, WP), _F32))


def _s2s_body(out_ref, bat_ref, wih_ref, whh_ref, bih_ref, bhh_ref, o_ref):
    outv = out_ref[...][:, :D]
    onehot = (bat_ref[...] == lax.broadcasted_iota(jnp.int32, (N, B), 1)
              ).astype(_F32)
    wih = wih_ref[...]
    whh = whh_ref[...]
    bih = bih_ref[...]
    bhh = bhh_ref[...]
    qs = jnp.zeros((B, 2 * D), _F32)
    hh = jnp.zeros((B, D), _F32)
    cc = jnp.zeros((B, D), _F32)
    for _ in range(3):
        gates = (lax.dot_general(qs, wih, (((1,), (1,)), ((), ())),
                                 preferred_element_type=_F32) + bih[None, :]
                 + lax.dot_general(hh, whh, (((1,), (1,)), ((), ())),
                                   preferred_element_type=_F32) + bhh[None, :])
        ig = jax.nn.sigmoid(gates[:, 0:D])
        fg = jax.nn.sigmoid(gates[:, D:2 * D])
        gg = jnp.tanh(gates[:, 2 * D:3 * D])
        og = jax.nn.sigmoid(gates[:, 3 * D:4 * D])
        cc = fg * cc + ig * gg
        hh = og * jnp.tanh(cc)
        qb = jnp.dot(onehot, hh, preferred_element_type=_F32)
        e = jnp.sum(outv * qb, axis=1, keepdims=True)
        masked = jnp.where(onehot > 0.0, e, -jnp.inf)
        emax = jnp.max(masked, axis=0, keepdims=True)
        emax = jnp.where(jnp.isfinite(emax), emax, 0.0)
        emax_n = lax.dot_general(onehot, emax, (((1,), (1,)), ((), ())),
                                 preferred_element_type=_F32)
        a = jnp.exp(e - emax_n)
        asum = lax.dot_general(onehot, a, (((0,), (0,)), ((), ())),
                               preferred_element_type=_F32)
        asum_n = jnp.dot(onehot, asum, preferred_element_type=_F32)
        a = a / (asum_n + 1e-16)
        r_ = lax.dot_general(onehot, a * outv, (((0,), (0,)), ((), ())),
                             preferred_element_type=_F32)
        qs = jnp.concatenate([hh, r_], axis=1)
    o_ref[...] = qs


_s2s = pl.pallas_call(
    _s2s_body, out_shape=jax.ShapeDtypeStruct((B, 2 * D), _F32))


def kernel(x, edge_index, edge_attr, batch, lin0_w, lin0_b, mlp_w1, mlp_b1,
           mlp_w2, mlp_b2, conv_bias, gru_w_ih, gru_w_hh, gru_b_ih, gru_b_hh,
           lstm_w_ih, lstm_w_hh, lstm_b_ih, lstm_b_hh):
    src = edge_index[0].astype(jnp.int32)
    dst = edge_index[1].astype(jnp.int32)
    pad = EPAD - E
    sent = N + jnp.arange(pad, dtype=jnp.int32) % (NTBL - N)
    src_p = jnp.concatenate([src, jnp.zeros((pad,), jnp.int32)])
    dst_p = jnp.concatenate([dst, sent])
    ea_p = jnp.concatenate(
        [edge_attr.astype(_F32), jnp.zeros((pad, edge_attr.shape[1]), _F32)])
    EH = EPAD // 2
    CPWH = CPW // 2
    src_h = src_p.reshape(2, NW, CPWH, CH)
    dst_h = dst_p.reshape(2, NW, CPWH, CH)
    dst_full = dst_p.reshape(NW, CPW, CH)
    zeros_hbm = jnp.zeros((RPT, WP), _F32)
    ones_hbm = jnp.ones((CH, WP), _F32)

    deg_p = _make_scatter(CPW, True)(dst_full, ones_hbm, zeros_hbm)
    out = _lin0(x, lin0_w, lin0_b)
    for _ in range(3):
        parts = []
        for hf in range(2):
            xj = _gather(CPWH)(out, src_h[hf])
            msg = _msg(EH)(ea_p[hf * EH:(hf + 1) * EH], xj,
                           mlp_w1, mlp_b1, mlp_w2, mlp_b2)
            parts.append(_make_scatter(CPWH, False)(dst_h[hf], msg, zeros_hbm))
        out = _upd(parts[0], parts[1], deg_p, out, conv_bias, gru_w_ih,
                   gru_w_hh, gru_b_ih, gru_b_hh)
    bat2 = batch.astype(jnp.int32).reshape(N, 1)
    return _s2s(out, bat2, lstm_w_ih, lstm_w_hh, lstm_b_ih, lstm_b_hh)
